# Initial kernel scaffold; baseline (speedup 1.0000x reference)
import jax
import jax.numpy as jnp
from jax.experimental import pallas as pl


def kernel(xs, ys, primitives):
    raise NotImplementedError("write your pallas kernel here")


# speedup vs baseline: 1.8231x; 1.8231x over previous
import functools

import jax
import jax.numpy as jnp
from jax import lax
from jax.experimental import pallas as pl
from jax.experimental.pallas import tpu as pltpu
from jax.experimental.pallas import tpu_sc as plsc

DIM = 32
NH = 8
R = 2049
CH = 64
NCHUNK = 33
ROW = NH * DIM * DIM


def _table_body(prim_ref, out_ref, p1_ref, a_ref, b_ref):
    rc = pl.program_id(1)
    eye = jnp.eye(DIM, dtype=jnp.float32)

    @pl.when(rc == 0)
    def _init():
        P = prim_ref[0]
        A0 = (P - P.T) * (1.0 / 1024.0)
        term = eye
        result = eye
        for k in range(1, 17):
            term = jnp.dot(term, A0, preferred_element_type=jnp.float32) * (1.0 / k)
            result = result + term
        for _ in range(10):
            result = jnp.dot(result, result, preferred_element_type=jnp.float32)
        p1_ref[0:DIM, :] = eye
        b_ref[...] = result
        n = 1
        for _ in range(6):
            p1_ref[n * DIM:2 * n * DIM, :] = jnp.dot(
                p1_ref[0:n * DIM, :], b_ref[...],
                preferred_element_type=jnp.float32)
            b_ref[...] = jnp.dot(b_ref[...], b_ref[...],
                                 preferred_element_type=jnp.float32)
            n *= 2
        a_ref[...] = eye

    chunk = jnp.dot(p1_ref[...], a_ref[...], preferred_element_type=jnp.float32)
    out_ref[...] = chunk.reshape(CH, 1, DIM, DIM)
    a_ref[...] = jnp.dot(a_ref[...], b_ref[...], preferred_element_type=jnp.float32)


def _build_table(prim_axis):
    return pl.pallas_call(
        _table_body,
        grid=(NH, NCHUNK),
        in_specs=[pl.BlockSpec((1, DIM, DIM), lambda h, rc: (h, 0, 0))],
        out_specs=pl.BlockSpec((CH, 1, DIM, DIM), lambda h, rc: (rc, h, 0, 0)),
        out_shape=jax.ShapeDtypeStruct((R, NH, DIM, DIM), jnp.float32),
        scratch_shapes=[
            pltpu.VMEM((CH * DIM, DIM), jnp.float32),
            pltpu.VMEM((DIM, DIM), jnp.float32),
            pltpu.VMEM((DIM, DIM), jnp.float32),
        ],
    )(prim_axis)


def _gather(table, idx3):
    info = plsc.get_sparse_core_info()
    nc, ns = info.num_cores, info.num_subcores
    nw = nc * ns
    _, k_chunks, c_rows = idx3.shape
    per_w = k_chunks * c_rows
    n_out = nw * per_w
    mesh = plsc.VectorSubcoreMesh(core_axis_name="c", subcore_axis_name="s")

    @functools.partial(
        pl.kernel, mesh=mesh,
        out_type=jax.ShapeDtypeStruct((n_out, ROW), jnp.float32),
        scratch_types=[
            pltpu.VMEM((k_chunks, c_rows), jnp.int32),
            pltpu.VMEM((c_rows, ROW), jnp.float32),
            pltpu.SemaphoreType.DMA,
        ],
    )
    def gather_k(table_hbm, idx_hbm, out_hbm, idx_v, buf_v, sem):
        wid = lax.axis_index("s") * nc + lax.axis_index("c")
        pltpu.sync_copy(idx_hbm.at[wid], idx_v)

        def body(ci, carry):
            pltpu.async_copy(table_hbm.at[idx_v.at[ci]], buf_v, sem).wait()
            pltpu.sync_copy(
                buf_v, out_hbm.at[pl.ds(wid * per_w + ci * c_rows, c_rows)])
            return carry

        lax.fori_loop(0, k_chunks, body, 0)

    return gather_k(table, idx3)


def kernel(xs, ys, primitives):
    prim = primitives.reshape(2, NH, DIM, DIM)
    info = plsc.get_sparse_core_info()
    nw = info.num_cores * info.num_subcores
    c_rows = 8
    k_chunks = xs.size // (nw * c_rows)

    tx = _build_table(prim[0])
    out_x = _gather(tx.reshape(R, ROW), xs.reshape(nw, k_chunks, c_rows))
    ty = _build_table(prim[1])
    out_y = _gather(ty.reshape(R, ROW), ys.reshape(nw, k_chunks, c_rows))

    shp = (xs.shape[0], xs.shape[1], NH, DIM, DIM)
    return (out_x.reshape(shp), out_y.reshape(shp))


# speedup vs baseline: 2.8170x; 1.5452x over previous
import functools

import jax
import jax.numpy as jnp
from jax import lax
from jax.experimental import pallas as pl
from jax.experimental.pallas import tpu as pltpu
from jax.experimental.pallas import tpu_sc as plsc

DIM = 32
NH = 8
R = 2049
CH = 64
NCHUNK = 33
ROW = NH * DIM * DIM


PACK = 128 // DIM
SUBR = DIM // PACK


def _mm(x, y):
    return jnp.dot(x, y, preferred_element_type=jnp.float32)


def _table_body(prim_ref, out_ref, p1_ref, a_ref, b_ref):
    rc = pl.program_id(1)
    eye = jnp.eye(DIM, dtype=jnp.float32)

    @pl.when(rc == 0)
    def _init():
        P = prim_ref[0]
        A0 = (P - P.T) * (1.0 / 1024.0)
        term = eye
        result = eye
        for k in range(1, 17):
            term = _mm(term, A0) * (1.0 / k)
            result = result + term
        for _ in range(10):
            result = _mm(result, result)
        r2 = lax.broadcasted_iota(jnp.int32, (128, DIM), 0)
        c2 = lax.broadcasted_iota(jnp.int32, (128, DIM), 1)
        J = jnp.where(r2 % DIM == c2, 1.0, 0.0)
        u = lax.broadcasted_iota(jnp.int32, (128, 128), 0)
        v = lax.broadcasted_iota(jnp.int32, (128, 128), 1)
        mask = jnp.where(u // DIM == v // DIM, 1.0, 0.0)
        b_ref[...] = _mm(_mm(J, result), J.T) * mask
        s8 = lax.broadcasted_iota(jnp.int32, (SUBR, 128), 0)
        l8 = lax.broadcasted_iota(jnp.int32, (SUBR, 128), 1)
        p1_ref[0:SUBR, :] = jnp.where(l8 % DIM == PACK * s8 + l8 // DIM,
                                      1.0, 0.0)
        n = 1
        for _ in range(6):
            p1_ref[n * SUBR:2 * n * SUBR, :] = _mm(p1_ref[0:n * SUBR, :],
                                                   b_ref[...])
            b_ref[...] = _mm(b_ref[...], b_ref[...])
            n *= 2
        a_ref[...] = jnp.eye(128, dtype=jnp.float32)

    chunk = _mm(p1_ref[...], a_ref[...])
    out_ref[...] = chunk.reshape(CH, SUBR, 128)
    a_ref[...] = _mm(a_ref[...], b_ref[...])


def _build_table(prim_axis):
    return pl.pallas_call(
        _table_body,
        grid=(NH, NCHUNK),
        in_specs=[pl.BlockSpec((1, DIM, DIM), lambda h, rc: (h, 0, 0))],
        out_specs=pl.BlockSpec((CH, SUBR, 128), lambda h, rc: (rc, h, 0)),
        out_shape=jax.ShapeDtypeStruct((R, ROW // 128, 128), jnp.float32),
        scratch_shapes=[
            pltpu.VMEM((CH * SUBR, 128), jnp.float32),
            pltpu.VMEM((128, 128), jnp.float32),
            pltpu.VMEM((128, 128), jnp.float32),
        ],
    )(prim_axis)


def _gather(table, idx3):
    info = plsc.get_sparse_core_info()
    nc, ns = info.num_cores, info.num_subcores
    nw = nc * ns
    _, k_chunks, c_rows = idx3.shape
    per_w = k_chunks * c_rows
    n_out = nw * per_w
    mesh = plsc.VectorSubcoreMesh(core_axis_name="c", subcore_axis_name="s")

    @functools.partial(
        pl.kernel, mesh=mesh,
        out_type=jax.ShapeDtypeStruct((n_out, ROW // 128, 128), jnp.float32),
        scratch_types=[
            pltpu.VMEM((k_chunks, c_rows), jnp.int32),
            pltpu.VMEM((c_rows, ROW // 128, 128), jnp.float32),
            pltpu.SemaphoreType.DMA,
        ],
    )
    def gather_k(table_hbm, idx_hbm, out_hbm, idx_v, buf_v, sem):
        wid = lax.axis_index("s") * nc + lax.axis_index("c")
        pltpu.sync_copy(idx_hbm.at[wid], idx_v)

        def body(ci, carry):
            pltpu.async_copy(table_hbm.at[idx_v.at[ci]], buf_v, sem).wait()
            pltpu.sync_copy(
                buf_v, out_hbm.at[pl.ds(wid * per_w + ci * c_rows, c_rows)])
            return carry

        lax.fori_loop(0, k_chunks, body, 0)

    return gather_k(table, idx3)


def kernel(xs, ys, primitives):
    prim = primitives.reshape(2, NH, DIM, DIM)
    info = plsc.get_sparse_core_info()
    nw = info.num_cores * info.num_subcores
    c_rows = 8
    k_chunks = xs.size // (nw * c_rows)

    tx = _build_table(prim[0])
    out_x = _gather(tx, xs.reshape(nw, k_chunks, c_rows))
    ty = _build_table(prim[1])
    out_y = _gather(ty, ys.reshape(nw, k_chunks, c_rows))

    shp = (xs.shape[0], xs.shape[1], NH, DIM, DIM)
    return (out_x.reshape(shp), out_y.reshape(shp))


# speedup vs baseline: 4.5491x; 1.6148x over previous
import functools

import jax
import jax.numpy as jnp
from jax import lax
from jax.experimental import pallas as pl
from jax.experimental.pallas import tpu as pltpu
from jax.experimental.pallas import tpu_sc as plsc

DIM = 32
NH = 8
R = 2049
CH = 64
NCHUNK = 33
ROW = NH * DIM * DIM


PACK = 128 // DIM
SUBR = DIM // PACK


def _mm(x, y):
    return jnp.dot(x, y, preferred_element_type=jnp.float32)


def _table_body(prim_ref, out_ref, p1_ref, a_ref, b_ref):
    rc = pl.program_id(1)
    eye = jnp.eye(DIM, dtype=jnp.float32)

    @pl.when(rc == 0)
    def _init():
        P = prim_ref[0]
        A0 = (P - P.T) * (1.0 / 1024.0)
        term = eye
        result = eye
        for k in range(1, 17):
            term = _mm(term, A0) * (1.0 / k)
            result = result + term
        for _ in range(10):
            result = _mm(result, result)
        r2 = lax.broadcasted_iota(jnp.int32, (128, DIM), 0)
        c2 = lax.broadcasted_iota(jnp.int32, (128, DIM), 1)
        J = jnp.where(r2 % DIM == c2, 1.0, 0.0)
        u = lax.broadcasted_iota(jnp.int32, (128, 128), 0)
        v = lax.broadcasted_iota(jnp.int32, (128, 128), 1)
        mask = jnp.where(u // DIM == v // DIM, 1.0, 0.0)
        b_ref[...] = _mm(_mm(J, result), J.T) * mask
        s8 = lax.broadcasted_iota(jnp.int32, (SUBR, 128), 0)
        l8 = lax.broadcasted_iota(jnp.int32, (SUBR, 128), 1)
        p1_ref[0:SUBR, :] = jnp.where(l8 % DIM == PACK * s8 + l8 // DIM,
                                      1.0, 0.0)
        n = 1
        for _ in range(6):
            p1_ref[n * SUBR:2 * n * SUBR, :] = _mm(p1_ref[0:n * SUBR, :],
                                                   b_ref[...])
            b_ref[...] = _mm(b_ref[...], b_ref[...])
            n *= 2
        a_ref[...] = jnp.eye(128, dtype=jnp.float32)

    chunk = _mm(p1_ref[...], a_ref[...])
    out_ref[...] = chunk.reshape(CH, SUBR, 128)
    a_ref[...] = _mm(a_ref[...], b_ref[...])


def _build_table(prim_axis):
    return pl.pallas_call(
        _table_body,
        grid=(NH, NCHUNK),
        in_specs=[pl.BlockSpec((1, DIM, DIM), lambda h, rc: (h, 0, 0))],
        out_specs=pl.BlockSpec((CH, SUBR, 128), lambda h, rc: (rc, h, 0)),
        out_shape=jax.ShapeDtypeStruct((R, ROW // 128, 128), jnp.float32),
        scratch_shapes=[
            pltpu.VMEM((CH * SUBR, 128), jnp.float32),
            pltpu.VMEM((128, 128), jnp.float32),
            pltpu.VMEM((128, 128), jnp.float32),
        ],
    )(prim_axis)


def _gather(table, idx3):
    info = plsc.get_sparse_core_info()
    nc, ns = info.num_cores, info.num_subcores
    nw = nc * ns
    _, k_chunks, c_rows = idx3.shape
    per_w = k_chunks * c_rows
    n_out = nw * per_w
    mesh = plsc.VectorSubcoreMesh(core_axis_name="c", subcore_axis_name="s")

    @functools.partial(
        pl.kernel, mesh=mesh,
        out_type=jax.ShapeDtypeStruct((n_out, ROW // 128, 128), jnp.float32),
        scratch_types=[
            pltpu.VMEM((k_chunks, c_rows), jnp.int32),
            pltpu.VMEM((c_rows, ROW // 128, 128), jnp.float32),
            pltpu.SemaphoreType.DMA,
        ],
    )
    def gather_k(table_hbm, idx_hbm, out_hbm, idx_v, buf_v, sem):
        wid = lax.axis_index("s") * nc + lax.axis_index("c")
        pltpu.sync_copy(idx_hbm.at[wid], idx_v)

        def body(ci, carry):
            pltpu.async_copy(table_hbm.at[idx_v.at[ci]], buf_v, sem).wait()
            pltpu.sync_copy(
                buf_v, out_hbm.at[pl.ds(wid * per_w + ci * c_rows, c_rows)])
            return carry

        lax.fori_loop(0, k_chunks, body, 0)

    return gather_k(table, idx3)


def _transpose_body(in_ref, out_ref):
    out_ref[...] = jnp.swapaxes(in_ref[:, 0, 0, :], 0, 1)[None]


def _to_seq_minor(g, nb, n):
    return pl.pallas_call(
        _transpose_body,
        grid=(nb, ROW // 128),
        in_specs=[pl.BlockSpec((n, 1, 1, 128), lambda b, f: (b, f, 0, 0))],
        out_specs=pl.BlockSpec((1, 128, n), lambda b, f: (b, f, 0)),
        out_shape=jax.ShapeDtypeStruct((nb, ROW, n), jnp.float32),
    )(g.reshape(nb * n, ROW // 128, 1, 128))


def kernel(xs, ys, primitives):
    prim = primitives.reshape(2, NH, DIM, DIM)
    info = plsc.get_sparse_core_info()
    nw = info.num_cores * info.num_subcores
    c_rows = 8
    k_chunks = xs.size // (nw * c_rows)
    nb, n = xs.shape

    tx = _build_table(prim[0])
    gx = _gather(tx, xs.reshape(nw, k_chunks, c_rows))
    ty = _build_table(prim[1])
    gy = _gather(ty, ys.reshape(nw, k_chunks, c_rows))
    tposed = [_to_seq_minor(g, nb, n) for g in (gx, gy)]

    return tuple(
        t.reshape(nb, NH, DIM, DIM, n).transpose(0, 4, 1, 2, 3)
        for t in tposed)


# speedup vs baseline: 5.2206x; 1.1476x over previous
import functools

import jax
import jax.numpy as jnp
from jax import lax
from jax.experimental import pallas as pl
from jax.experimental.pallas import tpu as pltpu
from jax.experimental.pallas import tpu_sc as plsc

DIM = 32
NH = 8
R = 2049
CH = 128
NCHUNK = 17
ROW = NH * DIM * DIM


PACK = 128 // DIM
SUBR = DIM // PACK


def _mm(x, y):
    return jnp.dot(x, y, preferred_element_type=jnp.float32)


def _table_body(prim_ref, out_ref, p1_ref, a_ref, b_ref):
    rc = pl.program_id(1)
    eye = jnp.eye(DIM, dtype=jnp.float32)

    @pl.when(rc == 0)
    def _init():
        P = prim_ref[0]
        A0 = (P - P.T) * (1.0 / 1024.0)
        term = eye
        result = eye
        for k in range(1, 17):
            term = _mm(term, A0) * (1.0 / k)
            result = result + term
        for _ in range(10):
            result = _mm(result, result)
        r2 = lax.broadcasted_iota(jnp.int32, (128, DIM), 0)
        c2 = lax.broadcasted_iota(jnp.int32, (128, DIM), 1)
        J = jnp.where(r2 % DIM == c2, 1.0, 0.0)
        u = lax.broadcasted_iota(jnp.int32, (128, 128), 0)
        v = lax.broadcasted_iota(jnp.int32, (128, 128), 1)
        mask = jnp.where(u // DIM == v // DIM, 1.0, 0.0)
        b_ref[...] = _mm(_mm(J, result), J.T) * mask
        s8 = lax.broadcasted_iota(jnp.int32, (SUBR, 128), 0)
        l8 = lax.broadcasted_iota(jnp.int32, (SUBR, 128), 1)
        p1_ref[0:SUBR, :] = jnp.where(l8 % DIM == PACK * s8 + l8 // DIM,
                                      1.0, 0.0)
        n = 1
        while n < CH:
            p1_ref[n * SUBR:2 * n * SUBR, :] = _mm(p1_ref[0:n * SUBR, :],
                                                   b_ref[...])
            b_ref[...] = _mm(b_ref[...], b_ref[...])
            n *= 2
        a_ref[...] = jnp.eye(128, dtype=jnp.float32)

    chunk = _mm(p1_ref[...], a_ref[...])
    out_ref[...] = chunk.reshape(CH, SUBR, 128)
    a_ref[...] = _mm(a_ref[...], b_ref[...])


def _build_table(prim_axis):
    return pl.pallas_call(
        _table_body,
        grid=(NH, NCHUNK),
        in_specs=[pl.BlockSpec((1, DIM, DIM), lambda h, rc: (h, 0, 0))],
        out_specs=pl.BlockSpec((CH, SUBR, 128), lambda h, rc: (rc, h, 0)),
        out_shape=jax.ShapeDtypeStruct((R, ROW // 128, 128), jnp.float32),
        scratch_shapes=[
            pltpu.VMEM((CH * SUBR, 128), jnp.float32),
            pltpu.VMEM((128, 128), jnp.float32),
            pltpu.VMEM((128, 128), jnp.float32),
        ],
    )(prim_axis)


def _gather(table, idx3):
    info = plsc.get_sparse_core_info()
    nc, ns = info.num_cores, info.num_subcores
    nw = nc * ns
    _, k_chunks, c_rows = idx3.shape
    per_w = k_chunks * c_rows
    n_out = nw * per_w
    mesh = plsc.VectorSubcoreMesh(core_axis_name="c", subcore_axis_name="s")

    @functools.partial(
        pl.kernel, mesh=mesh,
        out_type=jax.ShapeDtypeStruct((n_out, ROW // 128, 128), jnp.float32),
        scratch_types=[
            pltpu.VMEM((k_chunks, c_rows), jnp.int32),
            pltpu.VMEM((c_rows, ROW // 128, 128), jnp.float32),
            pltpu.SemaphoreType.DMA,
        ],
    )
    def gather_k(table_hbm, idx_hbm, out_hbm, idx_v, buf_v, sem):
        wid = lax.axis_index("s") * nc + lax.axis_index("c")
        pltpu.sync_copy(idx_hbm.at[wid], idx_v)

        def body(ci, carry):
            pltpu.async_copy(table_hbm.at[idx_v.at[ci]], buf_v, sem).wait()
            pltpu.sync_copy(
                buf_v, out_hbm.at[pl.ds(wid * per_w + ci * c_rows, c_rows)])
            return carry

        lax.fori_loop(0, k_chunks, body, 0)

    return gather_k(table, idx3)


def _transpose_body(in_ref, out_ref):
    out_ref[...] = jnp.swapaxes(in_ref[:, 0, 0, :], 0, 1)[None]


def _to_seq_minor(g, nb, n):
    return pl.pallas_call(
        _transpose_body,
        grid=(nb, ROW // 128),
        in_specs=[pl.BlockSpec((n, 1, 1, 128), lambda b, f: (b, f, 0, 0))],
        out_specs=pl.BlockSpec((1, 128, n), lambda b, f: (b, f, 0)),
        out_shape=jax.ShapeDtypeStruct((nb, ROW, n), jnp.float32),
    )(g.reshape(nb * n, ROW // 128, 1, 128))


def kernel(xs, ys, primitives):
    prim = primitives.reshape(2, NH, DIM, DIM)
    info = plsc.get_sparse_core_info()
    nw = info.num_cores * info.num_subcores
    c_rows = 8
    k_chunks = xs.size // (nw * c_rows)
    nb, n = xs.shape

    tx = _build_table(prim[0])
    gx = _gather(tx, xs.reshape(nw, k_chunks, c_rows))
    ty = _build_table(prim[1])
    gy = _gather(ty, ys.reshape(nw, k_chunks, c_rows))
    tposed = [_to_seq_minor(g, nb, n) for g in (gx, gy)]

    return tuple(
        t.reshape(nb, NH, DIM, DIM, n).transpose(0, 4, 1, 2, 3)
        for t in tposed)


# speedup vs baseline: 5.5840x; 1.0696x over previous
import functools

import jax
import jax.numpy as jnp
from jax import lax
from jax.experimental import pallas as pl
from jax.experimental.pallas import tpu as pltpu
from jax.experimental.pallas import tpu_sc as plsc

DIM = 32
NH = 8
R = 2049
CH = 256
NCHUNK = 9
ROW = NH * DIM * DIM


PACK = 128 // DIM
SUBR = DIM // PACK


def _mm(x, y):
    return jnp.dot(x, y, preferred_element_type=jnp.float32)


def _table_body(prim_ref, out_ref, p1_ref, a_ref, b_ref):
    rc = pl.program_id(1)
    eye = jnp.eye(DIM, dtype=jnp.float32)

    @pl.when(rc == 0)
    def _init():
        P = prim_ref[0]
        A0 = (P - P.T) * (1.0 / 1024.0)
        term = eye
        result = eye
        for k in range(1, 17):
            term = _mm(term, A0) * (1.0 / k)
            result = result + term
        for _ in range(10):
            result = _mm(result, result)
        r2 = lax.broadcasted_iota(jnp.int32, (128, DIM), 0)
        c2 = lax.broadcasted_iota(jnp.int32, (128, DIM), 1)
        J = jnp.where(r2 % DIM == c2, 1.0, 0.0)
        u = lax.broadcasted_iota(jnp.int32, (128, 128), 0)
        v = lax.broadcasted_iota(jnp.int32, (128, 128), 1)
        mask = jnp.where(u // DIM == v // DIM, 1.0, 0.0)
        b_ref[...] = _mm(_mm(J, result), J.T) * mask
        s8 = lax.broadcasted_iota(jnp.int32, (SUBR, 128), 0)
        l8 = lax.broadcasted_iota(jnp.int32, (SUBR, 128), 1)
        p1_ref[0:SUBR, :] = jnp.where(l8 % DIM == PACK * s8 + l8 // DIM,
                                      1.0, 0.0)
        n = 1
        while n < CH:
            p1_ref[n * SUBR:2 * n * SUBR, :] = _mm(p1_ref[0:n * SUBR, :],
                                                   b_ref[...])
            b_ref[...] = _mm(b_ref[...], b_ref[...])
            n *= 2
        a_ref[...] = jnp.eye(128, dtype=jnp.float32)

    chunk = _mm(p1_ref[...], a_ref[...])
    out_ref[...] = chunk.reshape(CH, SUBR, 128)
    a_ref[...] = _mm(a_ref[...], b_ref[...])


def _build_table(prim_axis):
    return pl.pallas_call(
        _table_body,
        grid=(NH, NCHUNK),
        in_specs=[pl.BlockSpec((1, DIM, DIM), lambda h, rc: (h, 0, 0))],
        out_specs=pl.BlockSpec((CH, SUBR, 128), lambda h, rc: (rc, h, 0)),
        out_shape=jax.ShapeDtypeStruct((R, ROW // 128, 128), jnp.float32),
        scratch_shapes=[
            pltpu.VMEM((CH * SUBR, 128), jnp.float32),
            pltpu.VMEM((128, 128), jnp.float32),
            pltpu.VMEM((128, 128), jnp.float32),
        ],
    )(prim_axis)


def _gather(table, idx3):
    info = plsc.get_sparse_core_info()
    nc, ns = info.num_cores, info.num_subcores
    nw = nc * ns
    _, k_chunks, c_rows = idx3.shape
    per_w = k_chunks * c_rows
    n_out = nw * per_w
    mesh = plsc.VectorSubcoreMesh(core_axis_name="c", subcore_axis_name="s")

    @functools.partial(
        pl.kernel, mesh=mesh,
        out_type=jax.ShapeDtypeStruct((n_out, ROW // 128, 128), jnp.float32),
        scratch_types=[
            pltpu.VMEM((k_chunks, c_rows), jnp.int32),
            pltpu.VMEM((c_rows, ROW // 128, 128), jnp.float32),
            pltpu.VMEM((c_rows, ROW // 128, 128), jnp.float32),
            pltpu.SemaphoreType.DMA,
            pltpu.SemaphoreType.DMA,
            pltpu.SemaphoreType.DMA,
            pltpu.SemaphoreType.DMA,
        ],
    )
    def gather_k(table_hbm, idx_hbm, out_hbm, idx_v, buf0, buf1,
                 gsem0, gsem1, osem0, osem1):
        wid = lax.axis_index("s") * nc + lax.axis_index("c")
        pltpu.sync_copy(idx_hbm.at[wid], idx_v)
        bufs, gsems, osems = (buf0, buf1), (gsem0, gsem1), (osem0, osem1)
        gathers = [None, None]
        stores = [None, None]
        gathers[0] = pltpu.async_copy(table_hbm.at[idx_v.at[0]], bufs[0],
                                      gsems[0])
        for ci in range(k_chunks):
            p = ci % 2
            q = (ci + 1) % 2
            if ci + 1 < k_chunks:
                if stores[q] is not None:
                    stores[q].wait()
                gathers[q] = pltpu.async_copy(
                    table_hbm.at[idx_v.at[ci + 1]], bufs[q], gsems[q])
            gathers[p].wait()
            stores[p] = pltpu.async_copy(
                bufs[p],
                out_hbm.at[pl.ds(wid * per_w + ci * c_rows, c_rows)],
                osems[p])
        stores[0].wait()
        stores[1].wait()

    return gather_k(table, idx3)


def _transpose_body(in_ref, out_ref):
    out_ref[...] = jnp.swapaxes(in_ref[:, 0, 0, :], 0, 1)[None]


def _to_seq_minor(g, nb, n):
    return pl.pallas_call(
        _transpose_body,
        grid=(nb, ROW // 128),
        in_specs=[pl.BlockSpec((n, 1, 1, 128), lambda b, f: (b, f, 0, 0))],
        out_specs=pl.BlockSpec((1, 128, n), lambda b, f: (b, f, 0)),
        out_shape=jax.ShapeDtypeStruct((nb, ROW, n), jnp.float32),
    )(g.reshape(nb * n, ROW // 128, 1, 128))


def kernel(xs, ys, primitives):
    prim = primitives.reshape(2, NH, DIM, DIM)
    info = plsc.get_sparse_core_info()
    nw = info.num_cores * info.num_subcores
    c_rows = 4
    k_chunks = xs.size // (nw * c_rows)
    nb, n = xs.shape

    tx = _build_table(prim[0])
    gx = _gather(tx, xs.reshape(nw, k_chunks, c_rows))
    ty = _build_table(prim[1])
    gy = _gather(ty, ys.reshape(nw, k_chunks, c_rows))
    tposed = [_to_seq_minor(g, nb, n) for g in (gx, gy)]

    return tuple(
        t.reshape(nb, NH, DIM, DIM, n).transpose(0, 4, 1, 2, 3)
        for t in tposed)


# speedup vs baseline: 6.6690x; 1.1943x over previous
import functools

import jax
import jax.numpy as jnp
from jax import lax
from jax.experimental import pallas as pl
from jax.experimental.pallas import tpu as pltpu
from jax.experimental.pallas import tpu_sc as plsc

DIM = 32
NH = 8
R = 2049
CH = 256
NCHUNK = 9
ROW = NH * DIM * DIM


PACK = 128 // DIM
SUBR = DIM // PACK


def _mm(x, y):
    return jnp.dot(x, y, preferred_element_type=jnp.float32)


def _table_body(prim_ref, out_ref, p1_ref, a_ref, b_ref):
    rc = pl.program_id(1)
    eye = jnp.eye(DIM, dtype=jnp.float32)

    @pl.when(rc == 0)
    def _init():
        P = prim_ref[0]
        A0 = (P - P.T) * (1.0 / 1024.0)
        term = eye
        result = eye
        for k in range(1, 17):
            term = _mm(term, A0) * (1.0 / k)
            result = result + term
        for _ in range(10):
            result = _mm(result, result)
        r2 = lax.broadcasted_iota(jnp.int32, (128, DIM), 0)
        c2 = lax.broadcasted_iota(jnp.int32, (128, DIM), 1)
        J = jnp.where(r2 % DIM == c2, 1.0, 0.0)
        u = lax.broadcasted_iota(jnp.int32, (128, 128), 0)
        v = lax.broadcasted_iota(jnp.int32, (128, 128), 1)
        mask = jnp.where(u // DIM == v // DIM, 1.0, 0.0)
        b_ref[...] = _mm(_mm(J, result), J.T) * mask
        s8 = lax.broadcasted_iota(jnp.int32, (SUBR, 128), 0)
        l8 = lax.broadcasted_iota(jnp.int32, (SUBR, 128), 1)
        p1_ref[0:SUBR, :] = jnp.where(l8 % DIM == PACK * s8 + l8 // DIM,
                                      1.0, 0.0)
        n = 1
        while n < CH:
            p1_ref[n * SUBR:2 * n * SUBR, :] = _mm(p1_ref[0:n * SUBR, :],
                                                   b_ref[...])
            b_ref[...] = _mm(b_ref[...], b_ref[...])
            n *= 2
        a_ref[...] = jnp.eye(128, dtype=jnp.float32)

    chunk = _mm(p1_ref[...], a_ref[...])
    out_ref[...] = chunk.reshape(CH, SUBR, 128)
    a_ref[...] = _mm(a_ref[...], b_ref[...])


def _build_table(prim_axis):
    return pl.pallas_call(
        _table_body,
        grid=(NH, NCHUNK),
        in_specs=[pl.BlockSpec((1, DIM, DIM), lambda h, rc: (h, 0, 0))],
        out_specs=pl.BlockSpec((CH, SUBR, 128), lambda h, rc: (rc, h, 0)),
        out_shape=jax.ShapeDtypeStruct((R, ROW // 128, 128), jnp.float32),
        scratch_shapes=[
            pltpu.VMEM((CH * SUBR, 128), jnp.float32),
            pltpu.VMEM((128, 128), jnp.float32),
            pltpu.VMEM((128, 128), jnp.float32),
        ],
    )(prim_axis)


def _gather(table, idx3):
    info = plsc.get_sparse_core_info()
    nc, ns = info.num_cores, info.num_subcores
    nw = nc * ns
    _, k_chunks, c_rows = idx3.shape
    per_w = k_chunks * c_rows
    n_out = nw * per_w
    mesh = plsc.VectorSubcoreMesh(core_axis_name="c", subcore_axis_name="s")

    @functools.partial(
        pl.kernel, mesh=mesh,
        out_type=jax.ShapeDtypeStruct((n_out, ROW // 128, 128), jnp.float32),
        scratch_types=[
            pltpu.VMEM((k_chunks, c_rows), jnp.int32),
            pltpu.VMEM((c_rows, ROW // 128, 128), jnp.float32),
            pltpu.VMEM((c_rows, ROW // 128, 128), jnp.float32),
            pltpu.SemaphoreType.DMA,
            pltpu.SemaphoreType.DMA,
            pltpu.SemaphoreType.DMA,
            pltpu.SemaphoreType.DMA,
        ],
    )
    def gather_k(table_hbm, idx_hbm, out_hbm, idx_v, buf0, buf1,
                 gsem0, gsem1, osem0, osem1):
        wid = lax.axis_index("s") * nc + lax.axis_index("c")
        pltpu.sync_copy(idx_hbm.at[wid], idx_v)
        bufs, gsems, osems = (buf0, buf1), (gsem0, gsem1), (osem0, osem1)
        gathers = [None, None]
        stores = [None, None]
        gathers[0] = pltpu.async_copy(table_hbm.at[idx_v.at[0]], bufs[0],
                                      gsems[0])
        for ci in range(k_chunks):
            p = ci % 2
            q = (ci + 1) % 2
            if ci + 1 < k_chunks:
                if stores[q] is not None:
                    stores[q].wait()
                gathers[q] = pltpu.async_copy(
                    table_hbm.at[idx_v.at[ci + 1]], bufs[q], gsems[q])
            gathers[p].wait()
            stores[p] = pltpu.async_copy(
                bufs[p],
                out_hbm.at[pl.ds(wid * per_w + ci * c_rows, c_rows)],
                osems[p])
        stores[0].wait()
        stores[1].wait()

    return gather_k(table, idx3)


FB = 8


def _transpose_body(in_ref, out_ref):
    x = in_ref[...]
    n = x.shape[0]
    out_ref[...] = x.reshape(n, FB * 128).T.reshape(1, FB * 128, n)


def _to_seq_minor(g, nb, n):
    return pl.pallas_call(
        _transpose_body,
        grid=(nb, ROW // (FB * 128)),
        in_specs=[pl.BlockSpec((n, FB, 128), lambda b, f: (b, f, 0))],
        out_specs=pl.BlockSpec((1, FB * 128, n), lambda b, f: (b, f, 0)),
        out_shape=jax.ShapeDtypeStruct((nb, ROW, n), jnp.float32),
    )(g.reshape(nb * n, ROW // 128, 128))


def kernel(xs, ys, primitives):
    prim = primitives.reshape(2, NH, DIM, DIM)
    info = plsc.get_sparse_core_info()
    nw = info.num_cores * info.num_subcores
    c_rows = 4
    k_chunks = xs.size // (nw * c_rows)
    nb, n = xs.shape

    tx = _build_table(prim[0])
    gx = _gather(tx, xs.reshape(nw, k_chunks, c_rows))
    ty = _build_table(prim[1])
    gy = _gather(ty, ys.reshape(nw, k_chunks, c_rows))
    tposed = [_to_seq_minor(g, nb, n) for g in (gx, gy)]

    return tuple(
        t.reshape(nb, NH, DIM, DIM, n).transpose(0, 4, 1, 2, 3)
        for t in tposed)
